# baseline (speedup 1.0000x reference)
import numpy as np
import jax
import jax.numpy as jnp
from jax import lax
from jax.experimental import pallas as pl
from jax.experimental.pallas import tpu as pltpu

_N = 10000
_E = 320000
_D = 128
_NR2 = 32
_NEG = 33
_NP = 10240
_NB = 40
_RB = 256

_PERM = np.zeros(13 * _D, dtype=np.int32)
_PERM[:_D] = np.arange(_D)
for _s in range(4):
    for _j in range(3):
        for _c in range(_D):
            _PERM[_D + (_s * 3 + _j) * _D + _c] = _D + 12 * _c + 3 * _s + _j
_PERM = jnp.asarray(_PERM)


def _prep_body(q_ref, w_ref, b_ref, o_ref):
    o_ref[...] = jnp.dot(q_ref[...], w_ref[...],
                         preferred_element_type=jnp.float32) + b_ref[...]


def _scale_body(deg_ref, h_ref, sn_ref, isc_ref, rdeg_ref, cnt_ref):
    deg = deg_ref[...]
    lg = jnp.log(deg)
    smean = jnp.sum(lg) / float(_N)
    sn = lg / (smean + 1e-10)
    sn_ref[...] = sn
    isc_ref[...] = 1.0 / jnp.clip(sn, 0.01, None)
    rdeg_ref[...] = 1.0 / deg
    ids = (lax.broadcasted_iota(jnp.int32, (_NB, _RB), 0) * _RB
           + lax.broadcasted_iota(jnp.int32, (_NB, _RB), 1))
    cnt = jnp.zeros((_NB, _RB), jnp.float32)
    for j in range(_NEG):
        cnt += (ids == h_ref[0, j]).astype(jnp.float32)
    cnt_ref[...] = cnt


def _combine_body(x_ref, sum_ref, sq_ref, mx_ref, mn_ref, rdeg_ref,
                  sn_ref, isc_ref, cnt_ref, q_ref, wl_ref, bl_ref, o_ref):
    x = x_ref[...]
    bnd = cnt_ref[...] * q_ref[...]
    rdeg = rdeg_ref[...]
    mean = (sum_ref[...] + bnd) * rdeg
    sqm = (sq_ref[...] + bnd * bnd) * rdeg
    mx = jnp.maximum(mx_ref[...], bnd)
    mn = jnp.minimum(mn_ref[...], bnd)
    std = jnp.sqrt(jnp.clip(sqm - mean * mean, 1e-6, None))
    s1 = sn_ref[...]
    s2 = isc_ref[...]
    feats = jnp.concatenate(
        [x, mean, mean * s1, mean * s2, mx, mx * s1, mx * s2,
         mn, mn * s1, mn * s2, std, std * s1, std * s2], axis=1)
    out = jnp.dot(feats, wl_ref[...], preferred_element_type=jnp.float32)
    o_ref[...] = jnp.maximum(out + bl_ref[...], 0.0)


def _final_body(x_ref, t_ref, q_ref, wlin_ref, blin_ref, wm1_ref, bm1_ref,
                wm2_ref, bm2_ref, o_ref):
    rows = [x_ref[pl.ds(t_ref[0, j], 1), :] for j in range(_NEG)]
    rows.append(jnp.zeros((40 - _NEG, _D), jnp.float32))
    tails = jnp.concatenate(rows, axis=0)
    q = jnp.broadcast_to(q_ref[...], (40, _D))
    feat = jnp.concatenate([tails, q], axis=1)
    h = jnp.dot(feat, wlin_ref[...], preferred_element_type=jnp.float32)
    h = jnp.maximum(h + blin_ref[...], 0.0)
    h = jnp.dot(h, wm1_ref[...], preferred_element_type=jnp.float32)
    h = jnp.maximum(h + bm1_ref[...], 0.0)
    s = jnp.dot(h, wm2_ref[...], preferred_element_type=jnp.float32)
    o_ref[...] = s + bm2_ref[...]


def _vmem(block, imap):
    return pl.BlockSpec(block, imap)


def _combine_call(xp, sums, sqs, mxs, mns, rdeg2, sn2, isc2, cnt2,
                  q1, wl_perm, bl):
    row = lambda i: (i, 0)
    const = lambda i: (0, 0)
    big = [_vmem((_RB, _D), row)] * 9
    specs = big + [_vmem((1, _D), const), _vmem((13 * _D, _D), const),
                   _vmem((1, _D), const)]
    return pl.pallas_call(
        _combine_body,
        grid=(_NB,),
        in_specs=specs,
        out_specs=_vmem((_RB, _D), row),
        out_shape=jax.ShapeDtypeStruct((_NP, _D), jnp.float32),
    )(xp, sums, sqs, mxs, mns, rdeg2, sn2, isc2, cnt2, q1, wl_perm, bl)


def kernel(h_index, r_index, t_index, hidden_states, rel_hidden_states,
           edge_index, edge_attr, score_text_embs, all_index,
           Wr0, br0, Wl0, bl0, Wr1, br1, Wl1, bl1,
           Wlin, blin, Wm1, bm1, Wm2, bm2):
    query = rel_hidden_states[r_index[0, 0]]
    q1 = query[None, :]

    q8 = jnp.broadcast_to(q1, (8, _D))
    wcat = jnp.concatenate([Wr0, Wr1], axis=1)
    bcat = jnp.concatenate([br0, br1])[None, :]
    prep = pl.pallas_call(
        _prep_body,
        in_specs=[pl.BlockSpec((8, _D), lambda: (0, 0)),
                  pl.BlockSpec((_D, 2 * _NR2 * _D), lambda: (0, 0)),
                  pl.BlockSpec((1, 2 * _NR2 * _D), lambda: (0, 0))],
        out_specs=pl.BlockSpec((8, 2 * _NR2 * _D), lambda: (0, 0)),
        out_shape=jax.ShapeDtypeStruct((8, 2 * _NR2 * _D), jnp.float32),
    )(q8, wcat, bcat)
    rel0 = prep[0, :_NR2 * _D].reshape(_NR2, _D)
    rel1 = prep[0, _NR2 * _D:].reshape(_NR2, _D)

    x0 = hidden_states + score_text_embs

    src = edge_index[0]
    dst = edge_index[1]
    ones = jnp.ones((_E,), jnp.float32)
    deg = jax.ops.segment_sum(ones, dst, num_segments=_N) + 1.0

    deg_pad = jnp.concatenate([deg, jnp.ones((_NP - _N,), jnp.float32)])
    deg_pad = deg_pad.reshape(_NB, _RB)
    h_s = h_index.astype(jnp.int32)
    sn, isc, rdeg, cnt = pl.pallas_call(
        _scale_body,
        in_specs=[pl.BlockSpec((_NB, _RB), lambda: (0, 0)),
                  pl.BlockSpec(memory_space=pltpu.SMEM)],
        out_specs=[pl.BlockSpec((_NB, _RB), lambda: (0, 0))] * 4,
        out_shape=[jax.ShapeDtypeStruct((_NB, _RB), jnp.float32)] * 4,
    )(deg_pad, h_s)
    to2d = lambda a: jnp.broadcast_to(a.reshape(_NP, 1), (_NP, _D))
    sn2, isc2, rdeg2, cnt2 = to2d(sn), to2d(isc), to2d(rdeg), to2d(cnt)

    wl0p = jnp.take(Wl0, _PERM, axis=0)
    wl1p = jnp.take(Wl1, _PERM, axis=0)
    padr = lambda a: jnp.pad(a, ((0, _NP - _N), (0, 0)))

    x = x0
    for rel, wlp, bl in ((rel0, wl0p, bl0), (rel1, wl1p, bl1)):
        ef = rel[edge_attr]
        m = x[src] * ef
        sums = jax.ops.segment_sum(m, dst, num_segments=_N)
        sqs = jax.ops.segment_sum(m * m, dst, num_segments=_N)
        mxs = jax.ops.segment_max(m, dst, num_segments=_N)
        mns = jax.ops.segment_min(m, dst, num_segments=_N)
        xp = _combine_call(padr(x), padr(sums), padr(sqs), padr(mxs),
                           padr(mns), rdeg2, sn2, isc2, cnt2,
                           q1, wlp, bl[None, :])
        x = xp[:_N]

    t_s = t_index.astype(jnp.int32)
    wm2p = jnp.pad(Wm2, ((0, 0), (0, _D - 1)))
    bm2p = jnp.pad(bm2, (0, _D - 1))[None, :]
    out = pl.pallas_call(
        _final_body,
        in_specs=[pl.BlockSpec((_N, _D), lambda: (0, 0)),
                  pl.BlockSpec(memory_space=pltpu.SMEM),
                  pl.BlockSpec((1, _D), lambda: (0, 0)),
                  pl.BlockSpec((2 * _D, _D), lambda: (0, 0)),
                  pl.BlockSpec((1, _D), lambda: (0, 0)),
                  pl.BlockSpec((_D, 2 * _D), lambda: (0, 0)),
                  pl.BlockSpec((1, 2 * _D), lambda: (0, 0)),
                  pl.BlockSpec((2 * _D, _D), lambda: (0, 0)),
                  pl.BlockSpec((1, _D), lambda: (0, 0))],
        out_specs=pl.BlockSpec((40, _D), lambda: (0, 0)),
        out_shape=jax.ShapeDtypeStruct((40, _D), jnp.float32),
    )(x, t_s, q1, Wlin, blin[None, :], Wm1, bm1[None, :], wm2p, bm2p)
    return out[:_NEG, 0].reshape(1, _NEG)


# speedup vs baseline: 1.5773x; 1.5773x over previous
import functools
import numpy as np
import jax
import jax.numpy as jnp
from jax import lax
from jax.experimental import pallas as pl
from jax.experimental.pallas import tpu as pltpu
from jax.experimental.pallas import tpu_sc as plsc

_N = 10000
_E = 320000
_D = 128
_NR2 = 32
_NEG = 33
_NP = 10240
_NB = 40
_RB = 256

_WRK = 32
_OWN = _NP // _WRK
_CH = _OWN // 2
_SB = 2048
_NBLK = 158
_EP = _SB * _NBLK
_QCAP = 160
_FLUSH = 128

_PERM = np.zeros(13 * _D, dtype=np.int32)
_PERM[:_D] = np.arange(_D)
for _s in range(4):
    for _j in range(3):
        for _c in range(_D):
            _PERM[_D + (_s * 3 + _j) * _D + _c] = _D + 12 * _c + 3 * _s + _j


def _agg_body(x_hbm, epk_hbm, rel_hbm,
              sum_hbm, sq_hbm, mx_hbm, mn_hbm, deg_hbm,
              ebuf_a, ebuf_b, qd, qs, qa, rows, rel_v,
              acc_sum, acc_sq, acc_mx, acc_mn, acc_deg,
              sem_a, sem_b, gsem):
    wid = lax.axis_index("s") * 2 + lax.axis_index("c")
    iota = lax.iota(jnp.int32, 16)
    z16 = jnp.zeros((16,), jnp.float32)
    c0 = jnp.zeros((16,), jnp.int32)
    c1 = c0 + 1
    c2 = c0 + 2

    pltpu.sync_copy(rel_hbm, rel_v)

    def flush_edges(n):
        nb = (n + 15) // 16

        def fire(b, _):
            sv = plsc.load_gather(qs, [b * 16 + iota])
            pltpu.make_async_copy(x_hbm.at[sv],
                                  rows.at[pl.ds(b * 16, 16), :], gsem).start()
            return 0
        lax.fori_loop(0, nb, fire, 0)

        def drain(b, _):
            sv = plsc.load_gather(qs, [b * 16 + iota])
            pltpu.make_async_copy(x_hbm.at[sv],
                                  rows.at[pl.ds(b * 16, 16), :], gsem).wait()
            return 0
        lax.fori_loop(0, nb, drain, 0)

        def edge(i, _):
            si = c0 + i
            ldv = plsc.load_gather(qd, [si])
            atv = plsc.load_gather(qa, [si])
            plsc.addupdate_scatter(acc_deg, [ldv],
                                   jnp.ones((16,), jnp.float32),
                                   mask=iota == 0)
            for j in range(8):
                col = j * 16 + iota
                xr = plsc.load_gather(rows, [si, col])
                rl = plsc.load_gather(rel_v, [atv, col])
                msg = xr * rl
                plsc.addupdate_scatter(acc_sum, [ldv, col], msg)
                plsc.addupdate_scatter(acc_sq, [ldv, col], msg * msg)
                cm = plsc.load_gather(acc_mx, [ldv, col])
                plsc.store_scatter(acc_mx, [ldv, col], jnp.maximum(cm, msg))
                cn = plsc.load_gather(acc_mn, [ldv, col])
                plsc.store_scatter(acc_mn, [ldv, col], jnp.minimum(cn, msg))
            return 0
        lax.fori_loop(0, n, edge, 0)

    def flush128(cc):
        flush_edges(_FLUSH)
        td = qd[pl.ds(_FLUSH, 16)]
        ts = qs[pl.ds(_FLUSH, 16)]
        ta = qa[pl.ds(_FLUSH, 16)]
        qd[pl.ds(0, 16)] = td
        qs[pl.ds(0, 16)] = ts
        qa[pl.ds(0, 16)] = ta
        return cc - _FLUSH

    def scan_superblock(ebuf, c, lo):
        def grp(gi, cc):
            ridx = gi * 48 + iota * 3
            dstv = plsc.load_gather(ebuf, [ridx])
            srcv = plsc.load_gather(ebuf, [ridx + 1])
            attrv = plsc.load_gather(ebuf, [ridx + 2])
            ld = dstv - lo
            m = (ld >= 0) & (ld < _CH)
            plsc.store_compressed(qd.at[pl.ds(cc, 16)], ld, mask=m)
            plsc.store_compressed(qs.at[pl.ds(cc, 16)], srcv, mask=m)
            plsc.store_compressed(qa.at[pl.ds(cc, 16)], attrv, mask=m)
            cc = cc + jnp.sum(m.astype(jnp.int32))
            cc = lax.cond(cc >= _FLUSH, flush128, lambda v: v, cc)
            return cc
        return lax.fori_loop(0, _SB // 16, grp, c)

    def chunk_pass(p, _):
        lo = wid * _OWN + p * _CH

        def init_row(r, _):
            for j in range(8):
                sl = pl.ds(j * 16, 16)
                acc_sum[r, sl] = z16
                acc_sq[r, sl] = z16
                acc_mx[r, sl] = z16 - 3.0e38
                acc_mn[r, sl] = z16 + 3.0e38
            return 0
        lax.fori_loop(0, _CH, init_row, 0)

        def init_q(i, _):
            sl = pl.ds(i * 16, 16)
            acc_deg[sl] = z16
            qd[sl] = c0
            qs[sl] = c0
            qa[sl] = c0
            return 0
        lax.fori_loop(0, _CH // 16, init_q, 0)

        pltpu.make_async_copy(epk_hbm.at[pl.ds(0, 3 * _SB)], ebuf_a,
                              sem_a).start()

        def blk_pair(g, c):
            pltpu.make_async_copy(epk_hbm.at[pl.ds(0, 3 * _SB)], ebuf_a,
                                  sem_a).wait()
            pltpu.make_async_copy(
                epk_hbm.at[pl.ds((2 * g + 1) * 3 * _SB, 3 * _SB)],
                ebuf_b, sem_b).start()
            c = scan_superblock(ebuf_a, c, lo)
            pltpu.make_async_copy(epk_hbm.at[pl.ds(0, 3 * _SB)], ebuf_b,
                                  sem_b).wait()

            @pl.when(2 * g + 2 < _NBLK)
            def _():
                pltpu.make_async_copy(
                    epk_hbm.at[pl.ds((2 * g + 2) * 3 * _SB, 3 * _SB)],
                    ebuf_a, sem_a).start()
            c = scan_superblock(ebuf_b, c, lo)
            return c

        c = lax.fori_loop(0, _NBLK // 2, blk_pair, 0)
        flush_edges(c)

        pltpu.sync_copy(acc_sum, sum_hbm.at[pl.ds(lo, _CH), :])
        pltpu.sync_copy(acc_sq, sq_hbm.at[pl.ds(lo, _CH), :])
        pltpu.sync_copy(acc_mx, mx_hbm.at[pl.ds(lo, _CH), :])
        pltpu.sync_copy(acc_mn, mn_hbm.at[pl.ds(lo, _CH), :])
        pltpu.sync_copy(acc_deg, deg_hbm.at[pl.ds(lo, _CH)])
        return 0

    lax.fori_loop(0, 2, chunk_pass, 0)


def _agg_call(x_pad, epk, rel):
    mesh = plsc.VectorSubcoreMesh(core_axis_name="c", subcore_axis_name="s")
    f32 = jnp.float32
    fn = functools.partial(
        pl.kernel, mesh=mesh,
        compiler_params=pltpu.CompilerParams(needs_layout_passes=False),
        out_type=[jax.ShapeDtypeStruct((_NP, _D), f32),
                  jax.ShapeDtypeStruct((_NP, _D), f32),
                  jax.ShapeDtypeStruct((_NP, _D), f32),
                  jax.ShapeDtypeStruct((_NP, _D), f32),
                  jax.ShapeDtypeStruct((_NP,), f32)],
        scratch_types=[pltpu.VMEM((3 * _SB,), jnp.int32),
                       pltpu.VMEM((3 * _SB,), jnp.int32),
                       pltpu.VMEM((_QCAP,), jnp.int32),
                       pltpu.VMEM((_QCAP,), jnp.int32),
                       pltpu.VMEM((_QCAP,), jnp.int32),
                       pltpu.VMEM((_QCAP - 16, _D), f32),
                       pltpu.VMEM((_NR2, _D), f32),
                       pltpu.VMEM((_CH, _D), f32),
                       pltpu.VMEM((_CH, _D), f32),
                       pltpu.VMEM((_CH, _D), f32),
                       pltpu.VMEM((_CH, _D), f32),
                       pltpu.VMEM((_CH,), f32),
                       pltpu.SemaphoreType.DMA,
                       pltpu.SemaphoreType.DMA,
                       pltpu.SemaphoreType.DMA],
    )(_agg_body)
    return fn(x_pad, epk, rel)


def _prep_body(q_ref, w_ref, b_ref, o_ref):
    o_ref[...] = jnp.dot(q_ref[...], w_ref[...],
                         preferred_element_type=jnp.float32) + b_ref[...]


def _scale_body(deg_ref, h_ref, sn_ref, isc_ref, rdeg_ref, cnt_ref):
    deg = deg_ref[...]
    lg = jnp.log(deg)
    smean = jnp.sum(lg) / float(_N)
    sn = lg / (smean + 1e-10)
    sn_ref[...] = sn
    isc_ref[...] = 1.0 / jnp.clip(sn, 0.01, None)
    rdeg_ref[...] = 1.0 / deg
    ids = (lax.broadcasted_iota(jnp.int32, (_NB, _RB), 0) * _RB
           + lax.broadcasted_iota(jnp.int32, (_NB, _RB), 1))
    cnt = jnp.zeros((_NB, _RB), jnp.float32)
    for j in range(_NEG):
        cnt += (ids == h_ref[0, j]).astype(jnp.float32)
    cnt_ref[...] = cnt


def _combine_body(x_ref, sum_ref, sq_ref, mx_ref, mn_ref, rdeg_ref,
                  sn_ref, isc_ref, cnt_ref, q_ref, wl_ref, bl_ref, o_ref):
    x = x_ref[...]
    bnd = cnt_ref[...] * q_ref[...]
    rdeg = rdeg_ref[...]
    mean = (sum_ref[...] + bnd) * rdeg
    sqm = (sq_ref[...] + bnd * bnd) * rdeg
    mx = jnp.maximum(mx_ref[...], bnd)
    mn = jnp.minimum(mn_ref[...], bnd)
    std = jnp.sqrt(jnp.clip(sqm - mean * mean, 1e-6, None))
    s1 = sn_ref[...]
    s2 = isc_ref[...]
    feats = jnp.concatenate(
        [x, mean, mean * s1, mean * s2, mx, mx * s1, mx * s2,
         mn, mn * s1, mn * s2, std, std * s1, std * s2], axis=1)
    out = jnp.dot(feats, wl_ref[...], preferred_element_type=jnp.float32)
    o_ref[...] = jnp.maximum(out + bl_ref[...], 0.0)


def _final_body(x_ref, t_ref, q_ref, wlin_ref, blin_ref, wm1_ref, bm1_ref,
                wm2_ref, bm2_ref, o_ref):
    rows = [x_ref[pl.ds(t_ref[0, j], 1), :] for j in range(_NEG)]
    rows.append(jnp.zeros((40 - _NEG, _D), jnp.float32))
    tails = jnp.concatenate(rows, axis=0)
    q = jnp.broadcast_to(q_ref[...], (40, _D))
    feat = jnp.concatenate([tails, q], axis=1)
    h = jnp.dot(feat, wlin_ref[...], preferred_element_type=jnp.float32)
    h = jnp.maximum(h + blin_ref[...], 0.0)
    h = jnp.dot(h, wm1_ref[...], preferred_element_type=jnp.float32)
    h = jnp.maximum(h + bm1_ref[...], 0.0)
    s = jnp.dot(h, wm2_ref[...], preferred_element_type=jnp.float32)
    o_ref[...] = s + bm2_ref[...]


def _vmem(block, imap):
    return pl.BlockSpec(block, imap)


def _combine_call(xp, sums, sqs, mxs, mns, rdeg2, sn2, isc2, cnt2,
                  q1, wl_perm, bl):
    row = lambda i: (i, 0)
    const = lambda i: (0, 0)
    big = [_vmem((_RB, _D), row)] * 9
    specs = big + [_vmem((1, _D), const), _vmem((13 * _D, _D), const),
                   _vmem((1, _D), const)]
    return pl.pallas_call(
        _combine_body,
        grid=(_NB,),
        in_specs=specs,
        out_specs=_vmem((_RB, _D), row),
        out_shape=jax.ShapeDtypeStruct((_NP, _D), jnp.float32),
    )(xp, sums, sqs, mxs, mns, rdeg2, sn2, isc2, cnt2, q1, wl_perm, bl)


def kernel(h_index, r_index, t_index, hidden_states, rel_hidden_states,
           edge_index, edge_attr, score_text_embs, all_index,
           Wr0, br0, Wl0, bl0, Wr1, br1, Wl1, bl1,
           Wlin, blin, Wm1, bm1, Wm2, bm2):
    query = rel_hidden_states[r_index[0, 0]]
    q1 = query[None, :]

    q8 = jnp.broadcast_to(q1, (8, _D))
    wcat = jnp.concatenate([Wr0, Wr1], axis=1)
    bcat = jnp.concatenate([br0, br1])[None, :]
    prep = pl.pallas_call(
        _prep_body,
        in_specs=[pl.BlockSpec((8, _D), lambda: (0, 0)),
                  pl.BlockSpec((_D, 2 * _NR2 * _D), lambda: (0, 0)),
                  pl.BlockSpec((1, 2 * _NR2 * _D), lambda: (0, 0))],
        out_specs=pl.BlockSpec((8, 2 * _NR2 * _D), lambda: (0, 0)),
        out_shape=jax.ShapeDtypeStruct((8, 2 * _NR2 * _D), jnp.float32),
    )(q8, wcat, bcat)
    rel0 = prep[0, :_NR2 * _D].reshape(_NR2, _D)
    rel1 = prep[0, _NR2 * _D:].reshape(_NR2, _D)

    x0 = hidden_states + score_text_embs

    i32 = jnp.int32
    ei = jnp.stack([edge_index[1].astype(i32), edge_index[0].astype(i32),
                    edge_attr.astype(i32)], axis=1)
    pad = jnp.broadcast_to(jnp.array([[16000, 0, 0]], i32), (_EP - _E, 3))
    epk = jnp.concatenate([ei, pad], axis=0).reshape(-1)

    padr = lambda a: jnp.pad(a, ((0, _NP - _N), (0, 0)))
    xp = padr(x0)

    wl0p = jnp.take(Wl0, _PERM, axis=0)
    wl1p = jnp.take(Wl1, _PERM, axis=0)

    h_s = h_index.astype(i32)
    sn2 = isc2 = rdeg2 = cnt2 = None
    for li, (rel, wlp, bl) in enumerate(((rel0, wl0p, bl0),
                                         (rel1, wl1p, bl1))):
        sums, sqs, mxs, mns, degr = _agg_call(xp, epk, rel)
        if li == 0:
            deg_pad = (degr + 1.0).reshape(_NB, _RB)
            sn, isc, rdeg, cnt = pl.pallas_call(
                _scale_body,
                in_specs=[pl.BlockSpec((_NB, _RB), lambda: (0, 0)),
                          pl.BlockSpec(memory_space=pltpu.SMEM)],
                out_specs=[pl.BlockSpec((_NB, _RB), lambda: (0, 0))] * 4,
                out_shape=[jax.ShapeDtypeStruct((_NB, _RB), jnp.float32)] * 4,
            )(deg_pad, h_s)
            to2d = lambda a: jnp.broadcast_to(a.reshape(_NP, 1), (_NP, _D))
            sn2, isc2, rdeg2, cnt2 = to2d(sn), to2d(isc), to2d(rdeg), to2d(cnt)
        xp = _combine_call(xp, sums, sqs, mxs, mns, rdeg2, sn2, isc2, cnt2,
                           q1, wlp, bl[None, :])

    t_s = t_index.astype(i32)
    wm2p = jnp.pad(Wm2, ((0, 0), (0, _D - 1)))
    bm2p = jnp.pad(bm2, (0, _D - 1))[None, :]
    out = pl.pallas_call(
        _final_body,
        in_specs=[pl.BlockSpec((_N, _D), lambda: (0, 0)),
                  pl.BlockSpec(memory_space=pltpu.SMEM),
                  pl.BlockSpec((1, _D), lambda: (0, 0)),
                  pl.BlockSpec((2 * _D, _D), lambda: (0, 0)),
                  pl.BlockSpec((1, _D), lambda: (0, 0)),
                  pl.BlockSpec((_D, 2 * _D), lambda: (0, 0)),
                  pl.BlockSpec((1, 2 * _D), lambda: (0, 0)),
                  pl.BlockSpec((2 * _D, _D), lambda: (0, 0)),
                  pl.BlockSpec((1, _D), lambda: (0, 0))],
        out_specs=pl.BlockSpec((40, _D), lambda: (0, 0)),
        out_shape=jax.ShapeDtypeStruct((40, _D), jnp.float32),
    )(xp[:_N], t_s, q1, Wlin, blin[None, :], Wm1, bm1[None, :], wm2p, bm2p)
    return out[:_NEG, 0].reshape(1, _NEG)


# speedup vs baseline: 1.9730x; 1.2509x over previous
import functools
import numpy as np
import jax
import jax.numpy as jnp
from jax import lax
from jax.experimental import pallas as pl
from jax.experimental.pallas import tpu as pltpu
from jax.experimental.pallas import tpu_sc as plsc

_N = 10000
_E = 320000
_D = 128
_NR2 = 32
_NEG = 33
_NP = 10240
_NB = 40
_RB = 256

_WRK = 32
_OWN = _NP // _WRK
_CH = _OWN // 2
_SB = 2048
_NBLK = 158
_EP = _SB * _NBLK
_QCAP = 192
_FLUSH = 128
_KCHK = 32

_PERM = np.zeros(13 * _D, dtype=np.int32)
_PERM[:_D] = np.arange(_D)
for _s in range(4):
    for _j in range(3):
        for _c in range(_D):
            _PERM[_D + (_s * 3 + _j) * _D + _c] = _D + 12 * _c + 3 * _s + _j


def _agg_body(x_hbm, epk_hbm, rel_hbm,
              sum_hbm, sq_hbm, mx_hbm, mn_hbm, deg_hbm,
              ebuf_a, ebuf_b, qd, qs, qa, rows, rel_v,
              acc_sum, acc_sq, acc_mx, acc_mn, acc_deg,
              sem_a, sem_b, gsem):
    wid = lax.axis_index("s") * 2 + lax.axis_index("c")
    iota = lax.iota(jnp.int32, 16)
    z16 = jnp.zeros((16,), jnp.float32)
    c0 = jnp.zeros((16,), jnp.int32)
    c1 = c0 + 1
    c2 = c0 + 2

    pltpu.sync_copy(rel_hbm, rel_v)

    def flush_edges(n):
        nb = (n + 15) // 16

        def fire(b, _):
            sv = plsc.load_gather(qs, [b * 16 + iota])
            pltpu.make_async_copy(x_hbm.at[sv],
                                  rows.at[pl.ds(b * 16, 16), :], gsem).start()
            return 0
        lax.fori_loop(0, nb, fire, 0)

        def edge(i, _):
            si = c0 + i
            ldv = plsc.load_gather(qd, [si])
            atv = plsc.load_gather(qa, [si])
            plsc.addupdate_scatter(acc_deg, [ldv],
                                   jnp.ones((16,), jnp.float32),
                                   mask=iota == 0)
            for j in range(8):
                col = j * 16 + iota
                xr = plsc.load_gather(rows, [si, col])
                rl = plsc.load_gather(rel_v, [atv, col])
                msg = xr * rl
                plsc.addupdate_scatter(acc_sum, [ldv, col], msg)
                plsc.addupdate_scatter(acc_sq, [ldv, col], msg * msg)
                cm = plsc.load_gather(acc_mx, [ldv, col])
                plsc.store_scatter(acc_mx, [ldv, col], jnp.maximum(cm, msg))
                cn = plsc.load_gather(acc_mn, [ldv, col])
                plsc.store_scatter(acc_mn, [ldv, col], jnp.minimum(cn, msg))
            return 0

        def sub(b, _):
            sv = plsc.load_gather(qs, [b * 16 + iota])
            pltpu.make_async_copy(x_hbm.at[sv],
                                  rows.at[pl.ds(b * 16, 16), :], gsem).wait()
            lax.fori_loop(b * 16, jnp.minimum(n, b * 16 + 16), edge, 0)
            return 0
        lax.fori_loop(0, nb, sub, 0)

    def flush128(cc):
        flush_edges(_FLUSH)
        for t in range(4):
            sl_hi = pl.ds(_FLUSH + t * 16, 16)
            sl_lo = pl.ds(t * 16, 16)
            qd[sl_lo] = qd[sl_hi]
            qs[sl_lo] = qs[sl_hi]
            qa[sl_lo] = qa[sl_hi]
        return cc - _FLUSH

    def scan_superblock(ebuf, c, lo):
        def grp(gi, ccv):
            ridx = gi * 48 + iota * 3
            dstv = plsc.load_gather(ebuf, [ridx])
            ld = dstv - lo
            m = (ld >= 0) & (ld < _CH)
            srcv = plsc.load_gather(ebuf, [ridx + 1])
            attrv = plsc.load_gather(ebuf, [ridx + 2])
            pos = jnp.minimum(ccv + plsc.cumsum(m.astype(jnp.int32)) - 1,
                              _QCAP - 1)
            plsc.store_scatter(qd, [pos], ld, mask=m)
            plsc.store_scatter(qs, [pos], srcv, mask=m)
            plsc.store_scatter(qa, [pos], attrv, mask=m)
            return ccv + plsc.all_reduce_population_count(m)

        def chk(k, cs):
            ccv = c0 + cs
            ccv = lax.fori_loop(k * _KCHK, (k + 1) * _KCHK, grp, ccv)
            cs = jnp.max(ccv)
            return lax.cond(cs >= _FLUSH, flush128, lambda v: v, cs)

        return lax.fori_loop(0, _SB // 16 // _KCHK, chk, c)

    def chunk_pass(p, _):
        lo = wid * _OWN + p * _CH

        def init_row(r, _):
            for j in range(8):
                sl = pl.ds(j * 16, 16)
                acc_sum[r, sl] = z16
                acc_sq[r, sl] = z16
                acc_mx[r, sl] = z16 - 3.0e38
                acc_mn[r, sl] = z16 + 3.0e38
            return 0
        lax.fori_loop(0, _CH, init_row, 0)

        def init_deg(i, _):
            acc_deg[pl.ds(i * 16, 16)] = z16
            return 0
        lax.fori_loop(0, _CH // 16, init_deg, 0)

        def init_q(i, _):
            sl = pl.ds(i * 16, 16)
            qd[sl] = c0
            qs[sl] = c0
            qa[sl] = c0
            return 0
        lax.fori_loop(0, _QCAP // 16, init_q, 0)

        pltpu.make_async_copy(epk_hbm.at[pl.ds(0, 3 * _SB)], ebuf_a,
                              sem_a).start()

        def blk_pair(g, c):
            pltpu.make_async_copy(epk_hbm.at[pl.ds(0, 3 * _SB)], ebuf_a,
                                  sem_a).wait()
            pltpu.make_async_copy(
                epk_hbm.at[pl.ds((2 * g + 1) * 3 * _SB, 3 * _SB)],
                ebuf_b, sem_b).start()
            c = scan_superblock(ebuf_a, c, lo)
            pltpu.make_async_copy(epk_hbm.at[pl.ds(0, 3 * _SB)], ebuf_b,
                                  sem_b).wait()

            @pl.when(2 * g + 2 < _NBLK)
            def _():
                pltpu.make_async_copy(
                    epk_hbm.at[pl.ds((2 * g + 2) * 3 * _SB, 3 * _SB)],
                    ebuf_a, sem_a).start()
            c = scan_superblock(ebuf_b, c, lo)
            return c

        c = lax.fori_loop(0, _NBLK // 2, blk_pair, 0)
        flush_edges(c)

        pltpu.sync_copy(acc_sum, sum_hbm.at[pl.ds(lo, _CH), :])
        pltpu.sync_copy(acc_sq, sq_hbm.at[pl.ds(lo, _CH), :])
        pltpu.sync_copy(acc_mx, mx_hbm.at[pl.ds(lo, _CH), :])
        pltpu.sync_copy(acc_mn, mn_hbm.at[pl.ds(lo, _CH), :])
        pltpu.sync_copy(acc_deg, deg_hbm.at[pl.ds(lo, _CH)])
        return 0

    lax.fori_loop(0, 2, chunk_pass, 0)


def _agg_call(x_pad, epk, rel):
    mesh = plsc.VectorSubcoreMesh(core_axis_name="c", subcore_axis_name="s")
    f32 = jnp.float32
    fn = functools.partial(
        pl.kernel, mesh=mesh,
        compiler_params=pltpu.CompilerParams(needs_layout_passes=False),
        out_type=[jax.ShapeDtypeStruct((_NP, _D), f32),
                  jax.ShapeDtypeStruct((_NP, _D), f32),
                  jax.ShapeDtypeStruct((_NP, _D), f32),
                  jax.ShapeDtypeStruct((_NP, _D), f32),
                  jax.ShapeDtypeStruct((_NP,), f32)],
        scratch_types=[pltpu.VMEM((3 * _SB,), jnp.int32),
                       pltpu.VMEM((3 * _SB,), jnp.int32),
                       pltpu.VMEM((_QCAP,), jnp.int32),
                       pltpu.VMEM((_QCAP,), jnp.int32),
                       pltpu.VMEM((_QCAP,), jnp.int32),
                       pltpu.VMEM((_QCAP, _D), f32),
                       pltpu.VMEM((_NR2, _D), f32),
                       pltpu.VMEM((_CH, _D), f32),
                       pltpu.VMEM((_CH, _D), f32),
                       pltpu.VMEM((_CH, _D), f32),
                       pltpu.VMEM((_CH, _D), f32),
                       pltpu.VMEM((_CH,), f32),
                       pltpu.SemaphoreType.DMA,
                       pltpu.SemaphoreType.DMA,
                       pltpu.SemaphoreType.DMA],
    )(_agg_body)
    return fn(x_pad, epk, rel)


def _prep_body(q_ref, w_ref, b_ref, o_ref):
    o_ref[...] = jnp.dot(q_ref[...], w_ref[...],
                         preferred_element_type=jnp.float32) + b_ref[...]


def _scale_body(deg_ref, h_ref, sn_ref, isc_ref, rdeg_ref, cnt_ref):
    deg = deg_ref[...]
    lg = jnp.log(deg)
    smean = jnp.sum(lg) / float(_N)
    sn = lg / (smean + 1e-10)
    sn_ref[...] = sn
    isc_ref[...] = 1.0 / jnp.clip(sn, 0.01, None)
    rdeg_ref[...] = 1.0 / deg
    ids = (lax.broadcasted_iota(jnp.int32, (_NB, _RB), 0) * _RB
           + lax.broadcasted_iota(jnp.int32, (_NB, _RB), 1))
    cnt = jnp.zeros((_NB, _RB), jnp.float32)
    for j in range(_NEG):
        cnt += (ids == h_ref[0, j]).astype(jnp.float32)
    cnt_ref[...] = cnt


def _combine_body(x_ref, sum_ref, sq_ref, mx_ref, mn_ref, rdeg_ref,
                  sn_ref, isc_ref, cnt_ref, q_ref, wl_ref, bl_ref, o_ref):
    x = x_ref[...]
    bnd = cnt_ref[...] * q_ref[...]
    rdeg = rdeg_ref[...]
    mean = (sum_ref[...] + bnd) * rdeg
    sqm = (sq_ref[...] + bnd * bnd) * rdeg
    mx = jnp.maximum(mx_ref[...], bnd)
    mn = jnp.minimum(mn_ref[...], bnd)
    std = jnp.sqrt(jnp.clip(sqm - mean * mean, 1e-6, None))
    s1 = sn_ref[...]
    s2 = isc_ref[...]
    feats = jnp.concatenate(
        [x, mean, mean * s1, mean * s2, mx, mx * s1, mx * s2,
         mn, mn * s1, mn * s2, std, std * s1, std * s2], axis=1)
    out = jnp.dot(feats, wl_ref[...], preferred_element_type=jnp.float32)
    o_ref[...] = jnp.maximum(out + bl_ref[...], 0.0)


def _final_body(x_ref, t_ref, q_ref, wlin_ref, blin_ref, wm1_ref, bm1_ref,
                wm2_ref, bm2_ref, o_ref):
    rows = [x_ref[pl.ds(t_ref[0, j], 1), :] for j in range(_NEG)]
    rows.append(jnp.zeros((40 - _NEG, _D), jnp.float32))
    tails = jnp.concatenate(rows, axis=0)
    q = jnp.broadcast_to(q_ref[...], (40, _D))
    feat = jnp.concatenate([tails, q], axis=1)
    h = jnp.dot(feat, wlin_ref[...], preferred_element_type=jnp.float32)
    h = jnp.maximum(h + blin_ref[...], 0.0)
    h = jnp.dot(h, wm1_ref[...], preferred_element_type=jnp.float32)
    h = jnp.maximum(h + bm1_ref[...], 0.0)
    s = jnp.dot(h, wm2_ref[...], preferred_element_type=jnp.float32)
    o_ref[...] = s + bm2_ref[...]


def _vmem(block, imap):
    return pl.BlockSpec(block, imap)


def _combine_call(xp, sums, sqs, mxs, mns, rdeg2, sn2, isc2, cnt2,
                  q1, wl_perm, bl):
    row = lambda i: (i, 0)
    const = lambda i: (0, 0)
    big = [_vmem((_RB, _D), row)] * 9
    specs = big + [_vmem((1, _D), const), _vmem((13 * _D, _D), const),
                   _vmem((1, _D), const)]
    return pl.pallas_call(
        _combine_body,
        grid=(_NB,),
        in_specs=specs,
        out_specs=_vmem((_RB, _D), row),
        out_shape=jax.ShapeDtypeStruct((_NP, _D), jnp.float32),
    )(xp, sums, sqs, mxs, mns, rdeg2, sn2, isc2, cnt2, q1, wl_perm, bl)


def kernel(h_index, r_index, t_index, hidden_states, rel_hidden_states,
           edge_index, edge_attr, score_text_embs, all_index,
           Wr0, br0, Wl0, bl0, Wr1, br1, Wl1, bl1,
           Wlin, blin, Wm1, bm1, Wm2, bm2):
    query = rel_hidden_states[r_index[0, 0]]
    q1 = query[None, :]

    q8 = jnp.broadcast_to(q1, (8, _D))
    wcat = jnp.concatenate([Wr0, Wr1], axis=1)
    bcat = jnp.concatenate([br0, br1])[None, :]
    prep = pl.pallas_call(
        _prep_body,
        in_specs=[pl.BlockSpec((8, _D), lambda: (0, 0)),
                  pl.BlockSpec((_D, 2 * _NR2 * _D), lambda: (0, 0)),
                  pl.BlockSpec((1, 2 * _NR2 * _D), lambda: (0, 0))],
        out_specs=pl.BlockSpec((8, 2 * _NR2 * _D), lambda: (0, 0)),
        out_shape=jax.ShapeDtypeStruct((8, 2 * _NR2 * _D), jnp.float32),
    )(q8, wcat, bcat)
    rel0 = prep[0, :_NR2 * _D].reshape(_NR2, _D)
    rel1 = prep[0, _NR2 * _D:].reshape(_NR2, _D)

    x0 = hidden_states + score_text_embs

    i32 = jnp.int32
    ei = jnp.stack([edge_index[1].astype(i32), edge_index[0].astype(i32),
                    edge_attr.astype(i32)], axis=1)
    pad = jnp.broadcast_to(jnp.array([[16000, 0, 0]], i32), (_EP - _E, 3))
    epk = jnp.concatenate([ei, pad], axis=0).reshape(-1)

    padr = lambda a: jnp.pad(a, ((0, _NP - _N), (0, 0)))
    xp = padr(x0)

    wl0p = jnp.take(Wl0, _PERM, axis=0)
    wl1p = jnp.take(Wl1, _PERM, axis=0)

    h_s = h_index.astype(i32)
    sn2 = isc2 = rdeg2 = cnt2 = None
    for li, (rel, wlp, bl) in enumerate(((rel0, wl0p, bl0),
                                         (rel1, wl1p, bl1))):
        sums, sqs, mxs, mns, degr = _agg_call(xp, epk, rel)
        if li == 0:
            deg_pad = (degr + 1.0).reshape(_NB, _RB)
            sn, isc, rdeg, cnt = pl.pallas_call(
                _scale_body,
                in_specs=[pl.BlockSpec((_NB, _RB), lambda: (0, 0)),
                          pl.BlockSpec(memory_space=pltpu.SMEM)],
                out_specs=[pl.BlockSpec((_NB, _RB), lambda: (0, 0))] * 4,
                out_shape=[jax.ShapeDtypeStruct((_NB, _RB), jnp.float32)] * 4,
            )(deg_pad, h_s)
            to2d = lambda a: jnp.broadcast_to(a.reshape(_NP, 1), (_NP, _D))
            sn2, isc2, rdeg2, cnt2 = to2d(sn), to2d(isc), to2d(rdeg), to2d(cnt)
        xp = _combine_call(xp, sums, sqs, mxs, mns, rdeg2, sn2, isc2, cnt2,
                           q1, wlp, bl[None, :])

    t_s = t_index.astype(i32)
    wm2p = jnp.pad(Wm2, ((0, 0), (0, _D - 1)))
    bm2p = jnp.pad(bm2, (0, _D - 1))[None, :]
    out = pl.pallas_call(
        _final_body,
        in_specs=[pl.BlockSpec((_N, _D), lambda: (0, 0)),
                  pl.BlockSpec(memory_space=pltpu.SMEM),
                  pl.BlockSpec((1, _D), lambda: (0, 0)),
                  pl.BlockSpec((2 * _D, _D), lambda: (0, 0)),
                  pl.BlockSpec((1, _D), lambda: (0, 0)),
                  pl.BlockSpec((_D, 2 * _D), lambda: (0, 0)),
                  pl.BlockSpec((1, 2 * _D), lambda: (0, 0)),
                  pl.BlockSpec((2 * _D, _D), lambda: (0, 0)),
                  pl.BlockSpec((1, _D), lambda: (0, 0))],
        out_specs=pl.BlockSpec((40, _D), lambda: (0, 0)),
        out_shape=jax.ShapeDtypeStruct((40, _D), jnp.float32),
    )(xp[:_N], t_s, q1, Wlin, blin[None, :], Wm1, bm1[None, :], wm2p, bm2p)
    return out[:_NEG, 0].reshape(1, _NEG)


# speedup vs baseline: 2.4136x; 1.2233x over previous
import functools
import numpy as np
import jax
import jax.numpy as jnp
from jax import lax
from jax.experimental import pallas as pl
from jax.experimental.pallas import tpu as pltpu
from jax.experimental.pallas import tpu_sc as plsc

_N = 10000
_E = 320000
_D = 128
_NR2 = 32
_NEG = 33
_NP = 10240
_NB = 40
_RB = 256

_WRK = 32
_OWN = _NP // _WRK
_CH = _OWN // 2
_SB = 2048
_NBLK = 158
_EP = _SB * _NBLK
_QCAP = 192
_FLUSH = 128
_KCHK = 32

_PERM = np.zeros(13 * _D, dtype=np.int32)
_PERM[:_D] = np.arange(_D)
for _s in range(4):
    for _j in range(3):
        for _c in range(_D):
            _PERM[_D + (_s * 3 + _j) * _D + _c] = _D + 12 * _c + 3 * _s + _j


def _agg_body(x_hbm, epk_hbm, rel_hbm,
              sum_hbm, sq_hbm, mx_hbm, mn_hbm, deg_hbm,
              ebuf_a, ebuf_b, qd, qs, qa, rows, rel_v,
              acc_sum, acc_sq, acc_mx, acc_mn, acc_deg,
              sem_a, sem_b, gsem):
    wid = lax.axis_index("s") * 2 + lax.axis_index("c")
    iota = lax.iota(jnp.int32, 16)
    z16 = jnp.zeros((16,), jnp.float32)
    c0 = jnp.zeros((16,), jnp.int32)
    c1 = c0 + 1
    c2 = c0 + 2

    pltpu.sync_copy(rel_hbm, rel_v)

    def flush_edges(n):
        nb = (n + 15) // 16

        def fire(b, _):
            sv = plsc.load_gather(qs, [b * 16 + iota])
            pltpu.make_async_copy(x_hbm.at[sv],
                                  rows.at[pl.ds(b * 16, 16), :], gsem).start()
            return 0
        lax.fori_loop(0, nb, fire, 0)

        def edge(i, _):
            si = c0 + i
            ldv = plsc.load_gather(qd, [si])
            atv = plsc.load_gather(qa, [si])
            plsc.addupdate_scatter(acc_deg, [ldv],
                                   jnp.ones((16,), jnp.float32),
                                   mask=iota == 0)
            for j in range(8):
                col = j * 16 + iota
                xr = plsc.load_gather(rows, [si, col])
                rl = plsc.load_gather(rel_v, [atv, col])
                msg = xr * rl
                plsc.addupdate_scatter(acc_sum, [ldv, col], msg)
                plsc.addupdate_scatter(acc_sq, [ldv, col], msg * msg)
                cm = plsc.load_gather(acc_mx, [ldv, col])
                plsc.store_scatter(acc_mx, [ldv, col], jnp.maximum(cm, msg))
                cn = plsc.load_gather(acc_mn, [ldv, col])
                plsc.store_scatter(acc_mn, [ldv, col], jnp.minimum(cn, msg))
            return 0

        def sub(b, _):
            sv = plsc.load_gather(qs, [b * 16 + iota])
            pltpu.make_async_copy(x_hbm.at[sv],
                                  rows.at[pl.ds(b * 16, 16), :], gsem).wait()
            lax.fori_loop(b * 16, jnp.minimum(n, b * 16 + 16), edge, 0)
            return 0
        lax.fori_loop(0, nb, sub, 0)

    def flush128(cc):
        flush_edges(_FLUSH)
        for t in range(4):
            sl_hi = pl.ds(_FLUSH + t * 16, 16)
            sl_lo = pl.ds(t * 16, 16)
            qd[sl_lo] = qd[sl_hi]
            qs[sl_lo] = qs[sl_hi]
            qa[sl_lo] = qa[sl_hi]
        return cc - _FLUSH

    def scan_superblock(ebuf, c, lo):
        def grp(gi, ccv):
            ridx = gi * 48 + iota * 3
            dstv = plsc.load_gather(ebuf, [ridx])
            ld = dstv - lo
            m = (ld >= 0) & (ld < _CH)
            srcv = plsc.load_gather(ebuf, [ridx + 1])
            attrv = plsc.load_gather(ebuf, [ridx + 2])
            pos = jnp.minimum(ccv + plsc.cumsum(m.astype(jnp.int32)) - 1,
                              _QCAP - 1)
            plsc.store_scatter(qd, [pos], ld, mask=m)
            plsc.store_scatter(qs, [pos], srcv, mask=m)
            plsc.store_scatter(qa, [pos], attrv, mask=m)
            return ccv + plsc.all_reduce_population_count(m)

        def chk(k, cs):
            ccv = plsc.parallel_loop(k * _KCHK, (k + 1) * _KCHK,
                                     carry=c0 + cs, unroll=2)(grp)
            cs = jnp.max(ccv)
            return lax.cond(cs >= _FLUSH, flush128, lambda v: v, cs)

        return lax.fori_loop(0, _SB // 16 // _KCHK, chk, c)

    def chunk_pass(p, _):
        lo = wid * _OWN + p * _CH

        def init_row(r, _):
            for j in range(8):
                sl = pl.ds(j * 16, 16)
                acc_sum[r, sl] = z16
                acc_sq[r, sl] = z16
                acc_mx[r, sl] = z16 - 3.0e38
                acc_mn[r, sl] = z16 + 3.0e38
            return 0
        lax.fori_loop(0, _CH, init_row, 0)

        def init_deg(i, _):
            acc_deg[pl.ds(i * 16, 16)] = z16
            return 0
        lax.fori_loop(0, _CH // 16, init_deg, 0)

        def init_q(i, _):
            sl = pl.ds(i * 16, 16)
            qd[sl] = c0
            qs[sl] = c0
            qa[sl] = c0
            return 0
        lax.fori_loop(0, _QCAP // 16, init_q, 0)

        pltpu.make_async_copy(epk_hbm.at[pl.ds(0, 3 * _SB)], ebuf_a,
                              sem_a).start()

        def blk_pair(g, c):
            pltpu.make_async_copy(epk_hbm.at[pl.ds(0, 3 * _SB)], ebuf_a,
                                  sem_a).wait()
            pltpu.make_async_copy(
                epk_hbm.at[pl.ds((2 * g + 1) * 3 * _SB, 3 * _SB)],
                ebuf_b, sem_b).start()
            c = scan_superblock(ebuf_a, c, lo)
            pltpu.make_async_copy(epk_hbm.at[pl.ds(0, 3 * _SB)], ebuf_b,
                                  sem_b).wait()

            @pl.when(2 * g + 2 < _NBLK)
            def _():
                pltpu.make_async_copy(
                    epk_hbm.at[pl.ds((2 * g + 2) * 3 * _SB, 3 * _SB)],
                    ebuf_a, sem_a).start()
            c = scan_superblock(ebuf_b, c, lo)
            return c

        c = lax.fori_loop(0, _NBLK // 2, blk_pair, 0)
        flush_edges(c)

        pltpu.sync_copy(acc_sum, sum_hbm.at[pl.ds(lo, _CH), :])
        pltpu.sync_copy(acc_sq, sq_hbm.at[pl.ds(lo, _CH), :])
        pltpu.sync_copy(acc_mx, mx_hbm.at[pl.ds(lo, _CH), :])
        pltpu.sync_copy(acc_mn, mn_hbm.at[pl.ds(lo, _CH), :])
        pltpu.sync_copy(acc_deg, deg_hbm.at[pl.ds(lo, _CH)])
        return 0

    lax.fori_loop(0, 2, chunk_pass, 0)


def _agg_call(x_pad, epk, rel):
    mesh = plsc.VectorSubcoreMesh(core_axis_name="c", subcore_axis_name="s")
    f32 = jnp.float32
    fn = functools.partial(
        pl.kernel, mesh=mesh,
        compiler_params=pltpu.CompilerParams(needs_layout_passes=False),
        out_type=[jax.ShapeDtypeStruct((_NP, _D), f32),
                  jax.ShapeDtypeStruct((_NP, _D), f32),
                  jax.ShapeDtypeStruct((_NP, _D), f32),
                  jax.ShapeDtypeStruct((_NP, _D), f32),
                  jax.ShapeDtypeStruct((_NP,), f32)],
        scratch_types=[pltpu.VMEM((3 * _SB,), jnp.int32),
                       pltpu.VMEM((3 * _SB,), jnp.int32),
                       pltpu.VMEM((_QCAP,), jnp.int32),
                       pltpu.VMEM((_QCAP,), jnp.int32),
                       pltpu.VMEM((_QCAP,), jnp.int32),
                       pltpu.VMEM((_QCAP, _D), f32),
                       pltpu.VMEM((_NR2, _D), f32),
                       pltpu.VMEM((_CH, _D), f32),
                       pltpu.VMEM((_CH, _D), f32),
                       pltpu.VMEM((_CH, _D), f32),
                       pltpu.VMEM((_CH, _D), f32),
                       pltpu.VMEM((_CH,), f32),
                       pltpu.SemaphoreType.DMA,
                       pltpu.SemaphoreType.DMA,
                       pltpu.SemaphoreType.DMA],
    )(_agg_body)
    return fn(x_pad, epk, rel)


def _prep_body(q_ref, w_ref, b_ref, o_ref):
    o_ref[...] = jnp.dot(q_ref[...], w_ref[...],
                         preferred_element_type=jnp.float32) + b_ref[...]


def _scale_body(deg_ref, h_ref, sn_ref, isc_ref, rdeg_ref, cnt_ref):
    deg = deg_ref[...]
    lg = jnp.log(deg)
    smean = jnp.sum(lg) / float(_N)
    sn = lg / (smean + 1e-10)
    sn_ref[...] = sn
    isc_ref[...] = 1.0 / jnp.clip(sn, 0.01, None)
    rdeg_ref[...] = 1.0 / deg
    ids = (lax.broadcasted_iota(jnp.int32, (_NB, _RB), 0) * _RB
           + lax.broadcasted_iota(jnp.int32, (_NB, _RB), 1))
    cnt = jnp.zeros((_NB, _RB), jnp.float32)
    for j in range(_NEG):
        cnt += (ids == h_ref[0, j]).astype(jnp.float32)
    cnt_ref[...] = cnt


def _combine_body(x_ref, sum_ref, sq_ref, mx_ref, mn_ref, rdeg_ref,
                  sn_ref, isc_ref, cnt_ref, q_ref, wl_ref, bl_ref, o_ref):
    x = x_ref[...]
    bnd = cnt_ref[...] * q_ref[...]
    rdeg = rdeg_ref[...]
    mean = (sum_ref[...] + bnd) * rdeg
    sqm = (sq_ref[...] + bnd * bnd) * rdeg
    mx = jnp.maximum(mx_ref[...], bnd)
    mn = jnp.minimum(mn_ref[...], bnd)
    std = jnp.sqrt(jnp.clip(sqm - mean * mean, 1e-6, None))
    s1 = sn_ref[...]
    s2 = isc_ref[...]
    feats = jnp.concatenate(
        [x, mean, mean * s1, mean * s2, mx, mx * s1, mx * s2,
         mn, mn * s1, mn * s2, std, std * s1, std * s2], axis=1)
    out = jnp.dot(feats, wl_ref[...], preferred_element_type=jnp.float32)
    o_ref[...] = jnp.maximum(out + bl_ref[...], 0.0)


def _final_body(x_ref, t_ref, q_ref, wlin_ref, blin_ref, wm1_ref, bm1_ref,
                wm2_ref, bm2_ref, o_ref):
    rows = [x_ref[pl.ds(t_ref[0, j], 1), :] for j in range(_NEG)]
    rows.append(jnp.zeros((40 - _NEG, _D), jnp.float32))
    tails = jnp.concatenate(rows, axis=0)
    q = jnp.broadcast_to(q_ref[...], (40, _D))
    feat = jnp.concatenate([tails, q], axis=1)
    h = jnp.dot(feat, wlin_ref[...], preferred_element_type=jnp.float32)
    h = jnp.maximum(h + blin_ref[...], 0.0)
    h = jnp.dot(h, wm1_ref[...], preferred_element_type=jnp.float32)
    h = jnp.maximum(h + bm1_ref[...], 0.0)
    s = jnp.dot(h, wm2_ref[...], preferred_element_type=jnp.float32)
    o_ref[...] = s + bm2_ref[...]


def _vmem(block, imap):
    return pl.BlockSpec(block, imap)


def _combine_call(xp, sums, sqs, mxs, mns, rdeg2, sn2, isc2, cnt2,
                  q1, wl_perm, bl):
    row = lambda i: (i, 0)
    const = lambda i: (0, 0)
    big = [_vmem((_RB, _D), row)] * 9
    specs = big + [_vmem((1, _D), const), _vmem((13 * _D, _D), const),
                   _vmem((1, _D), const)]
    return pl.pallas_call(
        _combine_body,
        grid=(_NB,),
        in_specs=specs,
        out_specs=_vmem((_RB, _D), row),
        out_shape=jax.ShapeDtypeStruct((_NP, _D), jnp.float32),
    )(xp, sums, sqs, mxs, mns, rdeg2, sn2, isc2, cnt2, q1, wl_perm, bl)


def kernel(h_index, r_index, t_index, hidden_states, rel_hidden_states,
           edge_index, edge_attr, score_text_embs, all_index,
           Wr0, br0, Wl0, bl0, Wr1, br1, Wl1, bl1,
           Wlin, blin, Wm1, bm1, Wm2, bm2):
    query = rel_hidden_states[r_index[0, 0]]
    q1 = query[None, :]

    q8 = jnp.broadcast_to(q1, (8, _D))
    wcat = jnp.concatenate([Wr0, Wr1], axis=1)
    bcat = jnp.concatenate([br0, br1])[None, :]
    prep = pl.pallas_call(
        _prep_body,
        in_specs=[pl.BlockSpec((8, _D), lambda: (0, 0)),
                  pl.BlockSpec((_D, 2 * _NR2 * _D), lambda: (0, 0)),
                  pl.BlockSpec((1, 2 * _NR2 * _D), lambda: (0, 0))],
        out_specs=pl.BlockSpec((8, 2 * _NR2 * _D), lambda: (0, 0)),
        out_shape=jax.ShapeDtypeStruct((8, 2 * _NR2 * _D), jnp.float32),
    )(q8, wcat, bcat)
    rel0 = prep[0, :_NR2 * _D].reshape(_NR2, _D)
    rel1 = prep[0, _NR2 * _D:].reshape(_NR2, _D)

    x0 = hidden_states + score_text_embs

    i32 = jnp.int32
    ei = jnp.stack([edge_index[1].astype(i32), edge_index[0].astype(i32),
                    edge_attr.astype(i32)], axis=1)
    pad = jnp.broadcast_to(jnp.array([[16000, 0, 0]], i32), (_EP - _E, 3))
    epk = jnp.concatenate([ei, pad], axis=0).reshape(-1)

    padr = lambda a: jnp.pad(a, ((0, _NP - _N), (0, 0)))
    xp = padr(x0)

    wl0p = jnp.take(Wl0, _PERM, axis=0)
    wl1p = jnp.take(Wl1, _PERM, axis=0)

    h_s = h_index.astype(i32)
    sn2 = isc2 = rdeg2 = cnt2 = None
    for li, (rel, wlp, bl) in enumerate(((rel0, wl0p, bl0),
                                         (rel1, wl1p, bl1))):
        sums, sqs, mxs, mns, degr = _agg_call(xp, epk, rel)
        if li == 0:
            deg_pad = (degr + 1.0).reshape(_NB, _RB)
            sn, isc, rdeg, cnt = pl.pallas_call(
                _scale_body,
                in_specs=[pl.BlockSpec((_NB, _RB), lambda: (0, 0)),
                          pl.BlockSpec(memory_space=pltpu.SMEM)],
                out_specs=[pl.BlockSpec((_NB, _RB), lambda: (0, 0))] * 4,
                out_shape=[jax.ShapeDtypeStruct((_NB, _RB), jnp.float32)] * 4,
            )(deg_pad, h_s)
            to2d = lambda a: jnp.broadcast_to(a.reshape(_NP, 1), (_NP, _D))
            sn2, isc2, rdeg2, cnt2 = to2d(sn), to2d(isc), to2d(rdeg), to2d(cnt)
        xp = _combine_call(xp, sums, sqs, mxs, mns, rdeg2, sn2, isc2, cnt2,
                           q1, wlp, bl[None, :])

    t_s = t_index.astype(i32)
    wm2p = jnp.pad(Wm2, ((0, 0), (0, _D - 1)))
    bm2p = jnp.pad(bm2, (0, _D - 1))[None, :]
    out = pl.pallas_call(
        _final_body,
        in_specs=[pl.BlockSpec((_N, _D), lambda: (0, 0)),
                  pl.BlockSpec(memory_space=pltpu.SMEM),
                  pl.BlockSpec((1, _D), lambda: (0, 0)),
                  pl.BlockSpec((2 * _D, _D), lambda: (0, 0)),
                  pl.BlockSpec((1, _D), lambda: (0, 0)),
                  pl.BlockSpec((_D, 2 * _D), lambda: (0, 0)),
                  pl.BlockSpec((1, 2 * _D), lambda: (0, 0)),
                  pl.BlockSpec((2 * _D, _D), lambda: (0, 0)),
                  pl.BlockSpec((1, _D), lambda: (0, 0))],
        out_specs=pl.BlockSpec((40, _D), lambda: (0, 0)),
        out_shape=jax.ShapeDtypeStruct((40, _D), jnp.float32),
    )(xp[:_N], t_s, q1, Wlin, blin[None, :], Wm1, bm1[None, :], wm2p, bm2p)
    return out[:_NEG, 0].reshape(1, _NEG)


# speedup vs baseline: 3.6119x; 1.4964x over previous
import functools
import numpy as np
import jax
import jax.numpy as jnp
from jax import lax
from jax.experimental import pallas as pl
from jax.experimental.pallas import tpu as pltpu
from jax.experimental.pallas import tpu_sc as plsc

_N = 10000
_E = 320000
_D = 128
_NR2 = 32
_NEG = 33
_NP = 10240
_NB = 40
_RB = 256

_WRK = 32
_OWN = _NP // _WRK
_CH = _OWN // 2
_SB = 2048
_NBLK = 158
_EP = _SB * _NBLK
_QCAP = 192
_FLUSH = 128
_KCHK = 32

_PERM = np.zeros(13 * _D, dtype=np.int32)
_PERM[:_D] = np.arange(_D)
for _s in range(4):
    for _j in range(3):
        for _c in range(_D):
            _PERM[_D + (_s * 3 + _j) * _D + _c] = _D + 12 * _c + 3 * _s + _j


def _agg_body(x_hbm, epk_hbm, rel_hbm,
              sum_hbm, sq_hbm, mx_hbm, mn_hbm, deg_hbm,
              ebuf_a, ebuf_b, qd, qs, qa, rows, rel_v,
              acc_sum, acc_sq, acc_mx, acc_mn, acc_deg,
              sem_a, sem_b, gsem):
    wid = lax.axis_index("s") * 2 + lax.axis_index("c")
    iota = lax.iota(jnp.int32, 16)
    z16 = jnp.zeros((16,), jnp.float32)
    c0 = jnp.zeros((16,), jnp.int32)
    c1 = c0 + 1
    c2 = c0 + 2

    pltpu.sync_copy(rel_hbm, rel_v)

    def flush_edges(n):
        nb = (n + 15) // 16

        def fire(b, _):
            sv = plsc.load_gather(qs, [b * 16 + iota])
            pltpu.make_async_copy(x_hbm.at[sv],
                                  rows.at[pl.ds(b * 16, 16), :], gsem).start()
            return 0
        lax.fori_loop(0, nb, fire, 0)

        def edge(i, _):
            si = c0 + i
            ldv = plsc.load_gather(qd, [si])
            atv = plsc.load_gather(qa, [si])
            lds = jnp.max(ldv)
            att = jnp.max(atv)
            plsc.addupdate_scatter(acc_deg, [ldv],
                                   jnp.ones((16,), jnp.float32),
                                   mask=iota == 0)
            for j in range(8):
                sl = pl.ds(j * 16, 16)
                msg = rows[i, sl] * rel_v[att, sl]
                plsc.addupdate(acc_sum.at[lds, sl], msg)
                plsc.addupdate(acc_sq.at[lds, sl], msg * msg)
                acc_mx[lds, sl] = jnp.maximum(acc_mx[lds, sl], msg)
                acc_mn[lds, sl] = jnp.minimum(acc_mn[lds, sl], msg)
            return 0

        def sub(b, _):
            sv = plsc.load_gather(qs, [b * 16 + iota])
            pltpu.make_async_copy(x_hbm.at[sv],
                                  rows.at[pl.ds(b * 16, 16), :], gsem).wait()
            lax.fori_loop(b * 16, jnp.minimum(n, b * 16 + 16), edge, 0)
            return 0
        lax.fori_loop(0, nb, sub, 0)

    def flush128(cc):
        flush_edges(_FLUSH)
        for t in range(4):
            sl_hi = pl.ds(_FLUSH + t * 16, 16)
            sl_lo = pl.ds(t * 16, 16)
            qd[sl_lo] = qd[sl_hi]
            qs[sl_lo] = qs[sl_hi]
            qa[sl_lo] = qa[sl_hi]
        return cc - _FLUSH

    def scan_superblock(ebuf, c, lo):
        def grp(gi, ccv):
            ridx = gi * 48 + iota * 3
            dstv = plsc.load_gather(ebuf, [ridx])
            ld = dstv - lo
            m = (ld >= 0) & (ld < _CH)
            srcv = plsc.load_gather(ebuf, [ridx + 1])
            attrv = plsc.load_gather(ebuf, [ridx + 2])
            pos = jnp.minimum(ccv + plsc.cumsum(m.astype(jnp.int32)) - 1,
                              _QCAP - 1)
            plsc.store_scatter(qd, [pos], ld, mask=m)
            plsc.store_scatter(qs, [pos], srcv, mask=m)
            plsc.store_scatter(qa, [pos], attrv, mask=m)
            return ccv + plsc.all_reduce_population_count(m)

        def chk(k, cs):
            ccv = plsc.parallel_loop(k * _KCHK, (k + 1) * _KCHK,
                                     carry=c0 + cs, unroll=2)(grp)
            cs = jnp.max(ccv)
            return lax.cond(cs >= _FLUSH, flush128, lambda v: v, cs)

        return lax.fori_loop(0, _SB // 16 // _KCHK, chk, c)

    def chunk_pass(p, _):
        lo = wid * _OWN + p * _CH

        def init_row(r, _):
            for j in range(8):
                sl = pl.ds(j * 16, 16)
                acc_sum[r, sl] = z16
                acc_sq[r, sl] = z16
                acc_mx[r, sl] = z16 - 3.0e38
                acc_mn[r, sl] = z16 + 3.0e38
            return 0
        lax.fori_loop(0, _CH, init_row, 0)

        def init_deg(i, _):
            acc_deg[pl.ds(i * 16, 16)] = z16
            return 0
        lax.fori_loop(0, _CH // 16, init_deg, 0)

        def init_q(i, _):
            sl = pl.ds(i * 16, 16)
            qd[sl] = c0
            qs[sl] = c0
            qa[sl] = c0
            return 0
        lax.fori_loop(0, _QCAP // 16, init_q, 0)

        pltpu.make_async_copy(epk_hbm.at[pl.ds(0, 3 * _SB)], ebuf_a,
                              sem_a).start()

        def blk_pair(g, c):
            pltpu.make_async_copy(epk_hbm.at[pl.ds(0, 3 * _SB)], ebuf_a,
                                  sem_a).wait()
            pltpu.make_async_copy(
                epk_hbm.at[pl.ds((2 * g + 1) * 3 * _SB, 3 * _SB)],
                ebuf_b, sem_b).start()
            c = scan_superblock(ebuf_a, c, lo)
            pltpu.make_async_copy(epk_hbm.at[pl.ds(0, 3 * _SB)], ebuf_b,
                                  sem_b).wait()

            @pl.when(2 * g + 2 < _NBLK)
            def _():
                pltpu.make_async_copy(
                    epk_hbm.at[pl.ds((2 * g + 2) * 3 * _SB, 3 * _SB)],
                    ebuf_a, sem_a).start()
            c = scan_superblock(ebuf_b, c, lo)
            return c

        c = lax.fori_loop(0, _NBLK // 2, blk_pair, 0)
        flush_edges(c)

        pltpu.sync_copy(acc_sum, sum_hbm.at[pl.ds(lo, _CH), :])
        pltpu.sync_copy(acc_sq, sq_hbm.at[pl.ds(lo, _CH), :])
        pltpu.sync_copy(acc_mx, mx_hbm.at[pl.ds(lo, _CH), :])
        pltpu.sync_copy(acc_mn, mn_hbm.at[pl.ds(lo, _CH), :])
        pltpu.sync_copy(acc_deg, deg_hbm.at[pl.ds(lo, _CH)])
        return 0

    lax.fori_loop(0, 2, chunk_pass, 0)


def _agg_call(x_pad, epk, rel):
    mesh = plsc.VectorSubcoreMesh(core_axis_name="c", subcore_axis_name="s")
    f32 = jnp.float32
    fn = functools.partial(
        pl.kernel, mesh=mesh,
        compiler_params=pltpu.CompilerParams(needs_layout_passes=False),
        out_type=[jax.ShapeDtypeStruct((_NP, _D), f32),
                  jax.ShapeDtypeStruct((_NP, _D), f32),
                  jax.ShapeDtypeStruct((_NP, _D), f32),
                  jax.ShapeDtypeStruct((_NP, _D), f32),
                  jax.ShapeDtypeStruct((_NP,), f32)],
        scratch_types=[pltpu.VMEM((3 * _SB,), jnp.int32),
                       pltpu.VMEM((3 * _SB,), jnp.int32),
                       pltpu.VMEM((_QCAP,), jnp.int32),
                       pltpu.VMEM((_QCAP,), jnp.int32),
                       pltpu.VMEM((_QCAP,), jnp.int32),
                       pltpu.VMEM((_QCAP, _D), f32),
                       pltpu.VMEM((_NR2, _D), f32),
                       pltpu.VMEM((_CH, _D), f32),
                       pltpu.VMEM((_CH, _D), f32),
                       pltpu.VMEM((_CH, _D), f32),
                       pltpu.VMEM((_CH, _D), f32),
                       pltpu.VMEM((_CH,), f32),
                       pltpu.SemaphoreType.DMA,
                       pltpu.SemaphoreType.DMA,
                       pltpu.SemaphoreType.DMA],
    )(_agg_body)
    return fn(x_pad, epk, rel)


def _prep_body(q_ref, w_ref, b_ref, o_ref):
    o_ref[...] = jnp.dot(q_ref[...], w_ref[...],
                         preferred_element_type=jnp.float32) + b_ref[...]


def _scale_body(deg_ref, h_ref, sn_ref, isc_ref, rdeg_ref, cnt_ref):
    deg = deg_ref[...]
    lg = jnp.log(deg)
    smean = jnp.sum(lg) / float(_N)
    sn = lg / (smean + 1e-10)
    sn_ref[...] = sn
    isc_ref[...] = 1.0 / jnp.clip(sn, 0.01, None)
    rdeg_ref[...] = 1.0 / deg
    ids = (lax.broadcasted_iota(jnp.int32, (_NB, _RB), 0) * _RB
           + lax.broadcasted_iota(jnp.int32, (_NB, _RB), 1))
    cnt = jnp.zeros((_NB, _RB), jnp.float32)
    for j in range(_NEG):
        cnt += (ids == h_ref[0, j]).astype(jnp.float32)
    cnt_ref[...] = cnt


def _combine_body(x_ref, sum_ref, sq_ref, mx_ref, mn_ref, rdeg_ref,
                  sn_ref, isc_ref, cnt_ref, q_ref, wl_ref, bl_ref, o_ref):
    x = x_ref[...]
    bnd = cnt_ref[...] * q_ref[...]
    rdeg = rdeg_ref[...]
    mean = (sum_ref[...] + bnd) * rdeg
    sqm = (sq_ref[...] + bnd * bnd) * rdeg
    mx = jnp.maximum(mx_ref[...], bnd)
    mn = jnp.minimum(mn_ref[...], bnd)
    std = jnp.sqrt(jnp.clip(sqm - mean * mean, 1e-6, None))
    s1 = sn_ref[...]
    s2 = isc_ref[...]
    feats = jnp.concatenate(
        [x, mean, mean * s1, mean * s2, mx, mx * s1, mx * s2,
         mn, mn * s1, mn * s2, std, std * s1, std * s2], axis=1)
    out = jnp.dot(feats, wl_ref[...], preferred_element_type=jnp.float32)
    o_ref[...] = jnp.maximum(out + bl_ref[...], 0.0)


def _final_body(x_ref, t_ref, q_ref, wlin_ref, blin_ref, wm1_ref, bm1_ref,
                wm2_ref, bm2_ref, o_ref):
    rows = [x_ref[pl.ds(t_ref[0, j], 1), :] for j in range(_NEG)]
    rows.append(jnp.zeros((40 - _NEG, _D), jnp.float32))
    tails = jnp.concatenate(rows, axis=0)
    q = jnp.broadcast_to(q_ref[...], (40, _D))
    feat = jnp.concatenate([tails, q], axis=1)
    h = jnp.dot(feat, wlin_ref[...], preferred_element_type=jnp.float32)
    h = jnp.maximum(h + blin_ref[...], 0.0)
    h = jnp.dot(h, wm1_ref[...], preferred_element_type=jnp.float32)
    h = jnp.maximum(h + bm1_ref[...], 0.0)
    s = jnp.dot(h, wm2_ref[...], preferred_element_type=jnp.float32)
    o_ref[...] = s + bm2_ref[...]


def _vmem(block, imap):
    return pl.BlockSpec(block, imap)


def _combine_call(xp, sums, sqs, mxs, mns, rdeg2, sn2, isc2, cnt2,
                  q1, wl_perm, bl):
    row = lambda i: (i, 0)
    const = lambda i: (0, 0)
    big = [_vmem((_RB, _D), row)] * 9
    specs = big + [_vmem((1, _D), const), _vmem((13 * _D, _D), const),
                   _vmem((1, _D), const)]
    return pl.pallas_call(
        _combine_body,
        grid=(_NB,),
        in_specs=specs,
        out_specs=_vmem((_RB, _D), row),
        out_shape=jax.ShapeDtypeStruct((_NP, _D), jnp.float32),
    )(xp, sums, sqs, mxs, mns, rdeg2, sn2, isc2, cnt2, q1, wl_perm, bl)


def kernel(h_index, r_index, t_index, hidden_states, rel_hidden_states,
           edge_index, edge_attr, score_text_embs, all_index,
           Wr0, br0, Wl0, bl0, Wr1, br1, Wl1, bl1,
           Wlin, blin, Wm1, bm1, Wm2, bm2):
    query = rel_hidden_states[r_index[0, 0]]
    q1 = query[None, :]

    q8 = jnp.broadcast_to(q1, (8, _D))
    wcat = jnp.concatenate([Wr0, Wr1], axis=1)
    bcat = jnp.concatenate([br0, br1])[None, :]
    prep = pl.pallas_call(
        _prep_body,
        in_specs=[pl.BlockSpec((8, _D), lambda: (0, 0)),
                  pl.BlockSpec((_D, 2 * _NR2 * _D), lambda: (0, 0)),
                  pl.BlockSpec((1, 2 * _NR2 * _D), lambda: (0, 0))],
        out_specs=pl.BlockSpec((8, 2 * _NR2 * _D), lambda: (0, 0)),
        out_shape=jax.ShapeDtypeStruct((8, 2 * _NR2 * _D), jnp.float32),
    )(q8, wcat, bcat)
    rel0 = prep[0, :_NR2 * _D].reshape(_NR2, _D)
    rel1 = prep[0, _NR2 * _D:].reshape(_NR2, _D)

    x0 = hidden_states + score_text_embs

    i32 = jnp.int32
    ei = jnp.stack([edge_index[1].astype(i32), edge_index[0].astype(i32),
                    edge_attr.astype(i32)], axis=1)
    pad = jnp.broadcast_to(jnp.array([[16000, 0, 0]], i32), (_EP - _E, 3))
    epk = jnp.concatenate([ei, pad], axis=0).reshape(-1)

    padr = lambda a: jnp.pad(a, ((0, _NP - _N), (0, 0)))
    xp = padr(x0)

    wl0p = jnp.take(Wl0, _PERM, axis=0)
    wl1p = jnp.take(Wl1, _PERM, axis=0)

    h_s = h_index.astype(i32)
    sn2 = isc2 = rdeg2 = cnt2 = None
    for li, (rel, wlp, bl) in enumerate(((rel0, wl0p, bl0),
                                         (rel1, wl1p, bl1))):
        sums, sqs, mxs, mns, degr = _agg_call(xp, epk, rel)
        if li == 0:
            deg_pad = (degr + 1.0).reshape(_NB, _RB)
            sn, isc, rdeg, cnt = pl.pallas_call(
                _scale_body,
                in_specs=[pl.BlockSpec((_NB, _RB), lambda: (0, 0)),
                          pl.BlockSpec(memory_space=pltpu.SMEM)],
                out_specs=[pl.BlockSpec((_NB, _RB), lambda: (0, 0))] * 4,
                out_shape=[jax.ShapeDtypeStruct((_NB, _RB), jnp.float32)] * 4,
            )(deg_pad, h_s)
            to2d = lambda a: jnp.broadcast_to(a.reshape(_NP, 1), (_NP, _D))
            sn2, isc2, rdeg2, cnt2 = to2d(sn), to2d(isc), to2d(rdeg), to2d(cnt)
        xp = _combine_call(xp, sums, sqs, mxs, mns, rdeg2, sn2, isc2, cnt2,
                           q1, wlp, bl[None, :])

    t_s = t_index.astype(i32)
    wm2p = jnp.pad(Wm2, ((0, 0), (0, _D - 1)))
    bm2p = jnp.pad(bm2, (0, _D - 1))[None, :]
    out = pl.pallas_call(
        _final_body,
        in_specs=[pl.BlockSpec((_N, _D), lambda: (0, 0)),
                  pl.BlockSpec(memory_space=pltpu.SMEM),
                  pl.BlockSpec((1, _D), lambda: (0, 0)),
                  pl.BlockSpec((2 * _D, _D), lambda: (0, 0)),
                  pl.BlockSpec((1, _D), lambda: (0, 0)),
                  pl.BlockSpec((_D, 2 * _D), lambda: (0, 0)),
                  pl.BlockSpec((1, 2 * _D), lambda: (0, 0)),
                  pl.BlockSpec((2 * _D, _D), lambda: (0, 0)),
                  pl.BlockSpec((1, _D), lambda: (0, 0))],
        out_specs=pl.BlockSpec((40, _D), lambda: (0, 0)),
        out_shape=jax.ShapeDtypeStruct((40, _D), jnp.float32),
    )(xp[:_N], t_s, q1, Wlin, blin[None, :], Wm1, bm1[None, :], wm2p, bm2p)
    return out[:_NEG, 0].reshape(1, _NEG)


# speedup vs baseline: 3.6162x; 1.0012x over previous
import functools
import numpy as np
import jax
import jax.numpy as jnp
from jax import lax
from jax.experimental import pallas as pl
from jax.experimental.pallas import tpu as pltpu
from jax.experimental.pallas import tpu_sc as plsc

_N = 10000
_E = 320000
_D = 128
_NR2 = 32
_NEG = 33
_NP = 10240
_NB = 40
_RB = 256

_WRK = 32
_OWN = _NP // _WRK
_CH = _OWN // 2
_SB = 2048
_NBLK = 158
_EP = _SB * _NBLK
_QCAP = 192
_FLUSH = 128
_KCHK = 32

_PERM = np.zeros(13 * _D, dtype=np.int32)
_PERM[:_D] = np.arange(_D)
for _s in range(4):
    for _j in range(3):
        for _c in range(_D):
            _PERM[_D + (_s * 3 + _j) * _D + _c] = _D + 12 * _c + 3 * _s + _j


def _agg_body(x_hbm, epk_hbm, rel_hbm,
              sum_hbm, sq_hbm, mx_hbm, mn_hbm, deg_hbm,
              ebuf_a, ebuf_b, qd, qs, qa, rows, rel_v,
              acc_sum, acc_sq, acc_mx, acc_mn, acc_deg,
              sem_a, sem_b, gsem):
    wid = lax.axis_index("s") * 2 + lax.axis_index("c")
    iota = lax.iota(jnp.int32, 16)
    z16 = jnp.zeros((16,), jnp.float32)
    c0 = jnp.zeros((16,), jnp.int32)
    c1 = c0 + 1
    c2 = c0 + 2

    pltpu.sync_copy(rel_hbm, rel_v)

    def flush_edges(n):
        nb = (n + 15) // 16

        def fire(b, _):
            sv = plsc.load_gather(qs, [b * 16 + iota])
            pltpu.make_async_copy(x_hbm.at[sv],
                                  rows.at[pl.ds(b * 16, 16), :], gsem).start()
            return 0
        lax.fori_loop(0, nb, fire, 0)

        def edge(i, _):
            si = c0 + i
            ldv = plsc.load_gather(qd, [si])
            atv = plsc.load_gather(qa, [si])
            lds = jnp.max(ldv)
            att = jnp.max(atv)
            plsc.addupdate_scatter(acc_deg, [ldv],
                                   jnp.ones((16,), jnp.float32),
                                   mask=iota == 0)
            for j in range(8):
                sl = pl.ds(j * 16, 16)
                msg = rows[i, sl] * rel_v[att, sl]
                plsc.addupdate(acc_sum.at[lds, sl], msg)
                plsc.addupdate(acc_sq.at[lds, sl], msg * msg)
                acc_mx[lds, sl] = jnp.maximum(acc_mx[lds, sl], msg)
                acc_mn[lds, sl] = jnp.minimum(acc_mn[lds, sl], msg)
            return 0

        def sub(b, _):
            sv = plsc.load_gather(qs, [b * 16 + iota])
            pltpu.make_async_copy(x_hbm.at[sv],
                                  rows.at[pl.ds(b * 16, 16), :], gsem).wait()
            lax.fori_loop(b * 16, jnp.minimum(n, b * 16 + 16), edge, 0)
            return 0
        lax.fori_loop(0, nb, sub, 0)

    def flush128(cc):
        flush_edges(_FLUSH)
        for t in range(4):
            sl_hi = pl.ds(_FLUSH + t * 16, 16)
            sl_lo = pl.ds(t * 16, 16)
            qd[sl_lo] = qd[sl_hi]
            qs[sl_lo] = qs[sl_hi]
            qa[sl_lo] = qa[sl_hi]
        return cc - _FLUSH

    def scan_superblock(ebuf, c, lo):
        def grp(gi, ccv):
            ridx = gi * 48 + iota * 3
            dstv = plsc.load_gather(ebuf, [ridx])
            ld = dstv - lo
            m = (ld >= 0) & (ld < _CH)
            srcv = plsc.load_gather(ebuf, [ridx + 1])
            attrv = plsc.load_gather(ebuf, [ridx + 2])
            pos = jnp.minimum(ccv + plsc.cumsum(m.astype(jnp.int32)) - 1,
                              _QCAP - 1)
            plsc.store_scatter(qd, [pos], ld, mask=m)
            plsc.store_scatter(qs, [pos], srcv, mask=m)
            plsc.store_scatter(qa, [pos], attrv, mask=m)
            return ccv + plsc.all_reduce_population_count(m)

        def chk(k, cs):
            ccv = plsc.parallel_loop(k * _KCHK, (k + 1) * _KCHK,
                                     carry=c0 + cs, unroll=4)(grp)
            cs = jnp.max(ccv)
            return lax.cond(cs >= _FLUSH, flush128, lambda v: v, cs)

        return lax.fori_loop(0, _SB // 16 // _KCHK, chk, c)

    def chunk_pass(p, _):
        lo = wid * _OWN + p * _CH

        def init_row(r):
            for j in range(8):
                sl = pl.ds(j * 16, 16)
                acc_sum[r, sl] = z16
                acc_sq[r, sl] = z16
                acc_mx[r, sl] = z16 - 3.0e38
                acc_mn[r, sl] = z16 + 3.0e38
        plsc.parallel_loop(0, _CH, unroll=2)(init_row)

        def init_deg(i, _):
            acc_deg[pl.ds(i * 16, 16)] = z16
            return 0
        lax.fori_loop(0, _CH // 16, init_deg, 0)

        def init_q(i, _):
            sl = pl.ds(i * 16, 16)
            qd[sl] = c0
            qs[sl] = c0
            qa[sl] = c0
            return 0
        lax.fori_loop(0, _QCAP // 16, init_q, 0)

        pltpu.make_async_copy(epk_hbm.at[pl.ds(0, 3 * _SB)], ebuf_a,
                              sem_a).start()

        def blk_pair(g, c):
            pltpu.make_async_copy(epk_hbm.at[pl.ds(0, 3 * _SB)], ebuf_a,
                                  sem_a).wait()
            pltpu.make_async_copy(
                epk_hbm.at[pl.ds((2 * g + 1) * 3 * _SB, 3 * _SB)],
                ebuf_b, sem_b).start()
            c = scan_superblock(ebuf_a, c, lo)
            pltpu.make_async_copy(epk_hbm.at[pl.ds(0, 3 * _SB)], ebuf_b,
                                  sem_b).wait()

            @pl.when(2 * g + 2 < _NBLK)
            def _():
                pltpu.make_async_copy(
                    epk_hbm.at[pl.ds((2 * g + 2) * 3 * _SB, 3 * _SB)],
                    ebuf_a, sem_a).start()
            c = scan_superblock(ebuf_b, c, lo)
            return c

        c = lax.fori_loop(0, _NBLK // 2, blk_pair, 0)
        flush_edges(c)

        pltpu.sync_copy(acc_sum, sum_hbm.at[pl.ds(lo, _CH), :])
        pltpu.sync_copy(acc_sq, sq_hbm.at[pl.ds(lo, _CH), :])
        pltpu.sync_copy(acc_mx, mx_hbm.at[pl.ds(lo, _CH), :])
        pltpu.sync_copy(acc_mn, mn_hbm.at[pl.ds(lo, _CH), :])
        pltpu.sync_copy(acc_deg, deg_hbm.at[pl.ds(lo, _CH)])
        return 0

    lax.fori_loop(0, 2, chunk_pass, 0)


def _agg_call(x_pad, epk, rel):
    mesh = plsc.VectorSubcoreMesh(core_axis_name="c", subcore_axis_name="s")
    f32 = jnp.float32
    fn = functools.partial(
        pl.kernel, mesh=mesh,
        compiler_params=pltpu.CompilerParams(needs_layout_passes=False),
        out_type=[jax.ShapeDtypeStruct((_NP, _D), f32),
                  jax.ShapeDtypeStruct((_NP, _D), f32),
                  jax.ShapeDtypeStruct((_NP, _D), f32),
                  jax.ShapeDtypeStruct((_NP, _D), f32),
                  jax.ShapeDtypeStruct((_NP,), f32)],
        scratch_types=[pltpu.VMEM((3 * _SB,), jnp.int32),
                       pltpu.VMEM((3 * _SB,), jnp.int32),
                       pltpu.VMEM((_QCAP,), jnp.int32),
                       pltpu.VMEM((_QCAP,), jnp.int32),
                       pltpu.VMEM((_QCAP,), jnp.int32),
                       pltpu.VMEM((_QCAP, _D), f32),
                       pltpu.VMEM((_NR2, _D), f32),
                       pltpu.VMEM((_CH, _D), f32),
                       pltpu.VMEM((_CH, _D), f32),
                       pltpu.VMEM((_CH, _D), f32),
                       pltpu.VMEM((_CH, _D), f32),
                       pltpu.VMEM((_CH,), f32),
                       pltpu.SemaphoreType.DMA,
                       pltpu.SemaphoreType.DMA,
                       pltpu.SemaphoreType.DMA],
    )(_agg_body)
    return fn(x_pad, epk, rel)


def _prep_body(q_ref, w_ref, b_ref, o_ref):
    o_ref[...] = jnp.dot(q_ref[...], w_ref[...],
                         preferred_element_type=jnp.float32) + b_ref[...]


def _scale_body(deg_ref, h_ref, sn_ref, isc_ref, rdeg_ref, cnt_ref):
    deg = deg_ref[...]
    lg = jnp.log(deg)
    smean = jnp.sum(lg) / float(_N)
    sn = lg / (smean + 1e-10)
    sn_ref[...] = sn
    isc_ref[...] = 1.0 / jnp.clip(sn, 0.01, None)
    rdeg_ref[...] = 1.0 / deg
    ids = (lax.broadcasted_iota(jnp.int32, (_NB, _RB), 0) * _RB
           + lax.broadcasted_iota(jnp.int32, (_NB, _RB), 1))
    cnt = jnp.zeros((_NB, _RB), jnp.float32)
    for j in range(_NEG):
        cnt += (ids == h_ref[0, j]).astype(jnp.float32)
    cnt_ref[...] = cnt


def _combine_body(x_ref, sum_ref, sq_ref, mx_ref, mn_ref, rdeg_ref,
                  sn_ref, isc_ref, cnt_ref, q_ref, wl_ref, bl_ref, o_ref):
    x = x_ref[...]
    bnd = cnt_ref[...] * q_ref[...]
    rdeg = rdeg_ref[...]
    mean = (sum_ref[...] + bnd) * rdeg
    sqm = (sq_ref[...] + bnd * bnd) * rdeg
    mx = jnp.maximum(mx_ref[...], bnd)
    mn = jnp.minimum(mn_ref[...], bnd)
    std = jnp.sqrt(jnp.clip(sqm - mean * mean, 1e-6, None))
    s1 = sn_ref[...]
    s2 = isc_ref[...]
    feats = jnp.concatenate(
        [x, mean, mean * s1, mean * s2, mx, mx * s1, mx * s2,
         mn, mn * s1, mn * s2, std, std * s1, std * s2], axis=1)
    out = jnp.dot(feats, wl_ref[...], preferred_element_type=jnp.float32)
    o_ref[...] = jnp.maximum(out + bl_ref[...], 0.0)


def _final_body(x_ref, t_ref, q_ref, wlin_ref, blin_ref, wm1_ref, bm1_ref,
                wm2_ref, bm2_ref, o_ref):
    rows = [x_ref[pl.ds(t_ref[0, j], 1), :] for j in range(_NEG)]
    rows.append(jnp.zeros((40 - _NEG, _D), jnp.float32))
    tails = jnp.concatenate(rows, axis=0)
    q = jnp.broadcast_to(q_ref[...], (40, _D))
    feat = jnp.concatenate([tails, q], axis=1)
    h = jnp.dot(feat, wlin_ref[...], preferred_element_type=jnp.float32)
    h = jnp.maximum(h + blin_ref[...], 0.0)
    h = jnp.dot(h, wm1_ref[...], preferred_element_type=jnp.float32)
    h = jnp.maximum(h + bm1_ref[...], 0.0)
    s = jnp.dot(h, wm2_ref[...], preferred_element_type=jnp.float32)
    o_ref[...] = s + bm2_ref[...]


def _vmem(block, imap):
    return pl.BlockSpec(block, imap)


def _combine_call(xp, sums, sqs, mxs, mns, rdeg2, sn2, isc2, cnt2,
                  q1, wl_perm, bl):
    row = lambda i: (i, 0)
    const = lambda i: (0, 0)
    big = [_vmem((_RB, _D), row)] * 9
    specs = big + [_vmem((1, _D), const), _vmem((13 * _D, _D), const),
                   _vmem((1, _D), const)]
    return pl.pallas_call(
        _combine_body,
        grid=(_NB,),
        in_specs=specs,
        out_specs=_vmem((_RB, _D), row),
        out_shape=jax.ShapeDtypeStruct((_NP, _D), jnp.float32),
    )(xp, sums, sqs, mxs, mns, rdeg2, sn2, isc2, cnt2, q1, wl_perm, bl)


def kernel(h_index, r_index, t_index, hidden_states, rel_hidden_states,
           edge_index, edge_attr, score_text_embs, all_index,
           Wr0, br0, Wl0, bl0, Wr1, br1, Wl1, bl1,
           Wlin, blin, Wm1, bm1, Wm2, bm2):
    query = rel_hidden_states[r_index[0, 0]]
    q1 = query[None, :]

    q8 = jnp.broadcast_to(q1, (8, _D))
    wcat = jnp.concatenate([Wr0, Wr1], axis=1)
    bcat = jnp.concatenate([br0, br1])[None, :]
    prep = pl.pallas_call(
        _prep_body,
        in_specs=[pl.BlockSpec((8, _D), lambda: (0, 0)),
                  pl.BlockSpec((_D, 2 * _NR2 * _D), lambda: (0, 0)),
                  pl.BlockSpec((1, 2 * _NR2 * _D), lambda: (0, 0))],
        out_specs=pl.BlockSpec((8, 2 * _NR2 * _D), lambda: (0, 0)),
        out_shape=jax.ShapeDtypeStruct((8, 2 * _NR2 * _D), jnp.float32),
    )(q8, wcat, bcat)
    rel0 = prep[0, :_NR2 * _D].reshape(_NR2, _D)
    rel1 = prep[0, _NR2 * _D:].reshape(_NR2, _D)

    x0 = hidden_states + score_text_embs

    i32 = jnp.int32
    ei = jnp.stack([edge_index[1].astype(i32), edge_index[0].astype(i32),
                    edge_attr.astype(i32)], axis=1)
    pad = jnp.broadcast_to(jnp.array([[16000, 0, 0]], i32), (_EP - _E, 3))
    epk = jnp.concatenate([ei, pad], axis=0).reshape(-1)

    padr = lambda a: jnp.pad(a, ((0, _NP - _N), (0, 0)))
    xp = padr(x0)

    wl0p = jnp.take(Wl0, _PERM, axis=0)
    wl1p = jnp.take(Wl1, _PERM, axis=0)

    h_s = h_index.astype(i32)
    sn2 = isc2 = rdeg2 = cnt2 = None
    for li, (rel, wlp, bl) in enumerate(((rel0, wl0p, bl0),
                                         (rel1, wl1p, bl1))):
        sums, sqs, mxs, mns, degr = _agg_call(xp, epk, rel)
        if li == 0:
            deg_pad = (degr + 1.0).reshape(_NB, _RB)
            sn, isc, rdeg, cnt = pl.pallas_call(
                _scale_body,
                in_specs=[pl.BlockSpec((_NB, _RB), lambda: (0, 0)),
                          pl.BlockSpec(memory_space=pltpu.SMEM)],
                out_specs=[pl.BlockSpec((_NB, _RB), lambda: (0, 0))] * 4,
                out_shape=[jax.ShapeDtypeStruct((_NB, _RB), jnp.float32)] * 4,
            )(deg_pad, h_s)
            to2d = lambda a: jnp.broadcast_to(a.reshape(_NP, 1), (_NP, _D))
            sn2, isc2, rdeg2, cnt2 = to2d(sn), to2d(isc), to2d(rdeg), to2d(cnt)
        xp = _combine_call(xp, sums, sqs, mxs, mns, rdeg2, sn2, isc2, cnt2,
                           q1, wlp, bl[None, :])

    t_s = t_index.astype(i32)
    wm2p = jnp.pad(Wm2, ((0, 0), (0, _D - 1)))
    bm2p = jnp.pad(bm2, (0, _D - 1))[None, :]
    out = pl.pallas_call(
        _final_body,
        in_specs=[pl.BlockSpec((_N, _D), lambda: (0, 0)),
                  pl.BlockSpec(memory_space=pltpu.SMEM),
                  pl.BlockSpec((1, _D), lambda: (0, 0)),
                  pl.BlockSpec((2 * _D, _D), lambda: (0, 0)),
                  pl.BlockSpec((1, _D), lambda: (0, 0)),
                  pl.BlockSpec((_D, 2 * _D), lambda: (0, 0)),
                  pl.BlockSpec((1, 2 * _D), lambda: (0, 0)),
                  pl.BlockSpec((2 * _D, _D), lambda: (0, 0)),
                  pl.BlockSpec((1, _D), lambda: (0, 0))],
        out_specs=pl.BlockSpec((40, _D), lambda: (0, 0)),
        out_shape=jax.ShapeDtypeStruct((40, _D), jnp.float32),
    )(xp[:_N], t_s, q1, Wlin, blin[None, :], Wm1, bm1[None, :], wm2p, bm2p)
    return out[:_NEG, 0].reshape(1, _NEG)
